# baseline (speedup 1.0000x reference)
import jax
import jax.numpy as jnp
from jax import lax
from jax.experimental import pallas as pl
from jax.experimental.pallas import tpu as pltpu
from jax.experimental.pallas import tpu_sc as plsc

B, C, H, W = 16, 64, 128, 128
HW = H * W
K = 500
NW = 32
PAIRS = 256
NIDX = PAIRS * C
KHI = K - PAIRS


def _sc_gather_kernel(outflat, ind2d, pred_hbm,
                      ind_v, idx_v, packed_v, rows_v, sem_g):
    cid = lax.axis_index("c")
    sid = lax.axis_index("s")
    row = cid * 16 + sid
    boff = sid * (C * HW)

    pltpu.sync_copy(ind2d.at[row], ind_v)

    lanes = lax.iota(jnp.int32, 16)
    coffs = [lanes * HW + (q * 16 * HW) for q in range(4)]

    def build(t, _):
        ivec = ind_v[pl.ds(t * 16, 16)] + boff
        for j in range(16):
            a = ivec[j]
            off = (16 * t + j) * C
            for q in range(4):
                idx_v[pl.ds(off + q * 16, 16)] = coffs[q] + a
        return 0
    lax.fori_loop(0, PAIRS // 16, build, 0)

    pltpu.async_copy(outflat.at[idx_v], packed_v, sem_g)
    pltpu.make_async_copy(outflat.at[idx_v], packed_v, sem_g).wait()

    def repack(p, _):
        for q in range(4):
            rows_v[p, pl.ds(q * 16, 16)] = packed_v[pl.ds(p * C + q * 16, 16)]
        return 0
    lax.fori_loop(0, PAIRS, repack, 0)

    pltpu.sync_copy(rows_v, pred_hbm.at[cid, sid])


def _tc_loss_kernel(pred_ref, mask_ref, targ_ref, out_ref, acc):
    b = pl.program_id(0)

    @pl.when(b == 0)
    def _():
        acc[0] = 0.0
        acc[1] = 0.0

    p0 = pred_ref[0, 0, :, :C]
    p1 = pred_ref[1, 0, :KHI, :C]
    m0 = mask_ref[0, :PAIRS, :]
    m1 = mask_ref[0, PAIRS:, :]
    t0 = targ_ref[0, :PAIRS, :]
    t1 = targ_ref[0, PAIRS:, :]
    s = jnp.sum(jnp.abs(p0 * m0 - t0 * m0)) + jnp.sum(jnp.abs(p1 * m1 - t1 * m1))
    acc[0] += s
    acc[1] += jnp.sum(m0) + jnp.sum(m1)

    @pl.when(b == B - 1)
    def _():
        out_ref[0, 0] = acc[0] / (acc[1] + 0.0001)


@jax.jit
def kernel(output, mask, ind, target):
    outflat = output.reshape(-1)
    lo = ind[:, :PAIRS]
    hi = jnp.pad(ind[:, PAIRS:], ((0, 0), (0, PAIRS - KHI)))
    ind2d = jnp.concatenate([lo, hi], axis=0)

    mesh = plsc.VectorSubcoreMesh(core_axis_name="c", subcore_axis_name="s")
    pred = pl.kernel(
        _sc_gather_kernel,
        mesh=mesh,
        out_type=jax.ShapeDtypeStruct((2, B, PAIRS, 128), jnp.float32),
        scratch_types=[
            pltpu.VMEM((PAIRS,), jnp.int32),
            pltpu.VMEM((NIDX,), jnp.int32),
            pltpu.VMEM((NIDX,), jnp.float32),
            pltpu.VMEM((PAIRS, 128), jnp.float32),
            pltpu.SemaphoreType.DMA,
        ],
    )(outflat, ind2d)

    loss2d = pl.pallas_call(
        _tc_loss_kernel,
        grid=(B,),
        in_specs=[
            pl.BlockSpec((2, 1, PAIRS, 128), lambda b: (0, b, 0, 0)),
            pl.BlockSpec((1, K, C), lambda b: (b, 0, 0)),
            pl.BlockSpec((1, K, C), lambda b: (b, 0, 0)),
        ],
        out_specs=pl.BlockSpec(
            (1, 1), lambda b: (0, 0), memory_space=pltpu.SMEM),
        out_shape=jax.ShapeDtypeStruct((1, 1), jnp.float32),
        scratch_shapes=[pltpu.SMEM((2,), jnp.float32)],
        compiler_params=pltpu.CompilerParams(
            dimension_semantics=("arbitrary",)),
    )(pred, mask, target)

    return loss2d[0, 0]


# speedup vs baseline: 1.1607x; 1.1607x over previous
import jax
import jax.numpy as jnp
from jax import lax
from jax.experimental import pallas as pl
from jax.experimental.pallas import tpu as pltpu
from jax.experimental.pallas import tpu_sc as plsc

B, C, H, W = 16, 64, 128, 128
HW = H * W
K = 500
NW = 32
PAIRS = 256
NIDX = PAIRS * C
KHI = K - PAIRS


def _sc_loss_kernel(outflat, ind2d, mask_in, targ_in, out_hbm,
                    ind_v, idx_v, pred_v, mask_v, targ_v, out_v,
                    sem_g, sem_m, sem_t):
    cid = lax.axis_index("c")
    sid = lax.axis_index("s")
    row = cid * 16 + sid
    boff = sid * (C * HW)

    pltpu.sync_copy(ind2d.at[row], ind_v)

    @pl.when(cid == 0)
    def _():
        pltpu.async_copy(mask_in.at[sid, pl.ds(0, PAIRS)],
                         mask_v.at[pl.ds(0, PAIRS)], sem_m)
        pltpu.async_copy(targ_in.at[sid, pl.ds(0, PAIRS)],
                         targ_v.at[pl.ds(0, PAIRS)], sem_t)

    @pl.when(cid == 1)
    def _():
        pltpu.async_copy(mask_in.at[sid, pl.ds(PAIRS, KHI)],
                         mask_v.at[pl.ds(0, KHI)], sem_m)
        pltpu.async_copy(targ_in.at[sid, pl.ds(PAIRS, KHI)],
                         targ_v.at[pl.ds(0, KHI)], sem_t)

    lanes = lax.iota(jnp.int32, 16)
    coffs = [lanes * HW + (q * 16 * HW) for q in range(4)]

    def build(t, _):
        ivec = ind_v[pl.ds(t * 16, 16)] + boff
        for j in range(16):
            a = ivec[j]
            off = (16 * t + j) * C
            for q in range(4):
                idx_v[pl.ds(off + q * 16, 16)] = coffs[q] + a
        return 0
    lax.fori_loop(0, PAIRS // 16, build, 0)

    pltpu.async_copy(outflat.at[idx_v], pred_v, sem_g)
    pltpu.make_async_copy(outflat.at[idx_v], pred_v, sem_g).wait()

    @pl.when(cid == 0)
    def _():
        pltpu.make_async_copy(mask_in.at[sid, pl.ds(0, PAIRS)],
                              mask_v.at[pl.ds(0, PAIRS)], sem_m).wait()
        pltpu.make_async_copy(targ_in.at[sid, pl.ds(0, PAIRS)],
                              targ_v.at[pl.ds(0, PAIRS)], sem_t).wait()

    @pl.when(cid == 1)
    def _():
        pltpu.make_async_copy(mask_in.at[sid, pl.ds(PAIRS, KHI)],
                              mask_v.at[pl.ds(0, KHI)], sem_m).wait()
        pltpu.make_async_copy(targ_in.at[sid, pl.ds(PAIRS, KHI)],
                              targ_v.at[pl.ds(0, KHI)], sem_t).wait()

    npairs = jnp.where(cid == 0, PAIRS, KHI)
    zero = jnp.zeros((16,), jnp.float32)

    def accum(r, carry):
        aabs, am = carry
        for q in range(4):
            sl = pl.ds(q * 16, 16)
            v = pred_v[pl.ds(r * C + q * 16, 16)]
            m = mask_v[r, sl]
            t = targ_v[r, sl]
            aabs = aabs + jnp.abs(v * m - t * m)
            am = am + m
        return (aabs, am)
    aabs, am = lax.fori_loop(0, npairs, accum, (zero, zero))

    out_v[pl.ds(0, 16)] = aabs
    out_v[pl.ds(16, 16)] = am
    pltpu.sync_copy(out_v, out_hbm.at[row])


@jax.jit
def kernel(output, mask, ind, target):
    outflat = output.reshape(-1)
    lo = ind[:, :PAIRS]
    hi = jnp.pad(ind[:, PAIRS:], ((0, 0), (0, PAIRS - KHI)))
    ind2d = jnp.concatenate([lo, hi], axis=0)

    mesh = plsc.VectorSubcoreMesh(core_axis_name="c", subcore_axis_name="s")
    partials = pl.kernel(
        _sc_loss_kernel,
        mesh=mesh,
        out_type=jax.ShapeDtypeStruct((NW, 32), jnp.float32),
        scratch_types=[
            pltpu.VMEM((PAIRS,), jnp.int32),
            pltpu.VMEM((NIDX,), jnp.int32),
            pltpu.VMEM((NIDX,), jnp.float32),
            pltpu.VMEM((PAIRS, C), jnp.float32),
            pltpu.VMEM((PAIRS, C), jnp.float32),
            pltpu.VMEM((32,), jnp.float32),
            pltpu.SemaphoreType.DMA,
            pltpu.SemaphoreType.DMA,
            pltpu.SemaphoreType.DMA,
        ],
    )(outflat, ind2d, mask, target)

    loss = jnp.sum(partials[:, :16]) / (jnp.sum(partials[:, 16:]) + 0.0001)
    return loss
